# baseline (speedup 1.0000x reference)
import functools

import jax
import jax.numpy as jnp
from jax import lax
from jax.experimental import pallas as pl
from jax.experimental.pallas import tpu as pltpu
from jax.experimental.pallas import tpu_sc as plsc

_N = 10000
_E = 320000
_C = 128
_H = 128
_NC = 2
_NS = 16
_NW = _NC * _NS
_K = 128
_NB = 2
_NCH = 82
_NGRP = _NCH // _NB
_EPAD = _NW * _K * _NCH
_NPAD = 10240
_RPT = _NPAD // _NS
_DUMMY = _N + 8

_mesh = plsc.VectorSubcoreMesh(
    core_axis_name="c", subcore_axis_name="s", num_cores=_NC, num_subcores=_NS
)


def _ring_pass(xh, srcr, wid, istg, idx_d, rows, acc, gsems, ssems, isems):
    pltpu.async_copy(srcr.at[wid].at[0], istg.at[0], isems[0])
    pltpu.async_copy(srcr.at[wid].at[1], istg.at[1], isems[1])
    pltpu.make_async_copy(srcr.at[wid].at[0], istg.at[0], isems[0]).wait()
    pltpu.async_copy(xh.at[istg.at[0]], rows[0], gsems[0])

    def grp(g, carry):
        for b in range(_NB):
            j = g * _NB + b
            nb = 1 - b

            def _issue_next():
                pltpu.make_async_copy(
                    srcr.at[wid].at[0], istg.at[nb], isems[nb]).wait()
                pltpu.async_copy(xh.at[istg.at[nb]], rows[nb], gsems[nb])

            if b == 0:
                _issue_next()
                pltpu.make_async_copy(
                    xh.at[pl.ds(0, _K)], rows[b], gsems[b]).wait()

                @pl.when(g < _NGRP - 1)
                def _():
                    pltpu.async_copy(
                        srcr.at[wid].at[j + 2], istg.at[b], isems[b])
            else:
                @pl.when(g < _NGRP - 1)
                def _():
                    _issue_next()

                pltpu.make_async_copy(
                    xh.at[pl.ds(0, _K)], rows[b], gsems[b]).wait()

                @pl.when(g < _NGRP - 1)
                def _():
                    pltpu.async_copy(
                        srcr.at[wid].at[j + 2], istg.at[b], isems[b])

            pltpu.sync_copy(rows[b], acc.at[idx_d.at[j]], add=True)
        return carry

    lax.fori_loop(0, _NGRP, grp, 0)


def _cnt_pass(ones_v, idx_d, acc, ssems):
    def chunk(j, carry):
        pltpu.sync_copy(ones_v, acc.at[idx_d.at[j]], add=True)
        return carry

    lax.fori_loop(0, _NCH, chunk, 0)


_SC_SCRATCH = [
    pltpu.VMEM((_NCH, _K), jnp.int32),
    pltpu.VMEM((_NB, _K), jnp.int32),
    pltpu.VMEM((_K, _C), jnp.float32),
    pltpu.VMEM((_K, _C), jnp.float32),
    pltpu.VMEM_SHARED((_NPAD, _C), jnp.float32),
    pltpu.SemaphoreType.DMA,
    pltpu.SemaphoreType.DMA,
    pltpu.SemaphoreType.DMA,
    pltpu.SemaphoreType.DMA,
    pltpu.SemaphoreType.DMA,
    pltpu.SemaphoreType.DMA,
]


def _spmm_enc(x0, x1, x2, x3, src_r, dst_r, zrow, ones_b):
    @functools.partial(
        pl.kernel,
        out_type=(
            jax.ShapeDtypeStruct((_NC, 4, _NPAD, _C), jnp.float32),
            jax.ShapeDtypeStruct((_NC, _NPAD, _C), jnp.float32),
        ),
        mesh=_mesh,
        scratch_types=_SC_SCRATCH,
    )
    def k(x0r, x1r, x2r, x3r, srcr, dstr, zr, ones_h, out_s, out_c,
          idx_d, istg, r0b, r1b, acc, g0, g1, s0, s1, i0, i1):
        c = lax.axis_index("c")
        s = lax.axis_index("s")
        wid = c * _NS + s
        r0 = s * _RPT
        rows = (r0b, r1b)
        gsems = (g0, g1)
        ssems = (s0, s1)
        isems = (i0, i1)
        pltpu.sync_copy(dstr.at[wid], idx_d)

        pltpu.sync_copy(ones_h, r0b)
        pltpu.sync_copy(zr, acc.at[pl.ds(r0, _RPT)])
        plsc.subcore_barrier()
        _cnt_pass(r0b, idx_d, acc, ssems)
        plsc.subcore_barrier()
        pltpu.sync_copy(acc.at[pl.ds(r0, _RPT)], out_c.at[c].at[pl.ds(r0, _RPT)])
        plsc.subcore_barrier()

        for t, xh in enumerate((x0r, x1r, x2r, x3r)):
            pltpu.sync_copy(zr, acc.at[pl.ds(r0, _RPT)])
            plsc.subcore_barrier()
            _ring_pass(xh, srcr, wid, istg, idx_d, rows, acc,
                       gsems, ssems, isems)
            plsc.subcore_barrier()
            pltpu.sync_copy(
                acc.at[pl.ds(r0, _RPT)], out_s.at[c].at[t].at[pl.ds(r0, _RPT)]
            )
            plsc.subcore_barrier()

    return k(x0, x1, x2, x3, src_r, dst_r, zrow, ones_b)


def _spmm_one(x, src_r, dst_r, zrow):
    @functools.partial(
        pl.kernel,
        out_type=jax.ShapeDtypeStruct((_NC, _NPAD, _C), jnp.float32),
        mesh=_mesh,
        scratch_types=_SC_SCRATCH,
    )
    def k(xh, srcr, dstr, zr, out_s,
          idx_d, istg, r0b, r1b, acc, g0, g1, s0, s1, i0, i1):
        c = lax.axis_index("c")
        s = lax.axis_index("s")
        wid = c * _NS + s
        r0 = s * _RPT
        pltpu.sync_copy(dstr.at[wid], idx_d)
        pltpu.sync_copy(zr, acc.at[pl.ds(r0, _RPT)])
        plsc.subcore_barrier()
        _ring_pass(xh, srcr, wid, istg, idx_d, (r0b, r1b), acc,
                   (g0, g1), (s0, s1), (i0, i1))
        plsc.subcore_barrier()
        pltpu.sync_copy(acc.at[pl.ds(r0, _RPT)], out_s.at[c].at[pl.ds(r0, _RPT)])

    return k(x, src_r, dst_r, zrow)


def _mm(a, b):
    return jnp.dot(a, b, preferred_element_type=jnp.float32)


def _gru(x, h, wi, wh, bi, bh):
    gi = _mm(x, wi) + bi
    gh = _mm(h, wh) + bh
    r = jax.nn.sigmoid(gi[:, :_H] + gh[:, :_H])
    z = jax.nn.sigmoid(gi[:, _H:2 * _H] + gh[:, _H:2 * _H])
    n = jnp.tanh(gi[:, 2 * _H:] + r * gh[:, 2 * _H:])
    return (1.0 - z) * n + z * h


_R = 1000


def _tc_enc(x_all, s_parts, cnt_parts, wl, wr, bl, gw, pw, pb):
    def body(x_ref, s_ref, cnt_ref, wl_ref, wr_ref, bl_ref,
             wi1, wh1, bi1, bh1, wi2, wh2, bi2, bh2, wi3, wh3, bi3, bh3,
             pw_ref, pb_ref, y1_ref, h1_ref, h2_ref, h3_ref):
        inv = 1.0 / jnp.clip(cnt_ref[0] + cnt_ref[1], 1.0, None)
        wlv, wrv, blv = wl_ref[...], wr_ref[...], bl_ref[...]
        g = (wi1[...], wh1[...], bi1[...], bh1[...],
             wi2[...], wh2[...], bi2[...], bh2[...],
             wi3[...], wh3[...], bi3[...], bh3[...])
        h1 = jnp.zeros((_R, _H), jnp.float32)
        h2 = jnp.zeros((_R, _H), jnp.float32)
        h3 = jnp.zeros((_R, _H), jnp.float32)
        xr = None
        for t in range(4):
            st = s_ref[0, t] + s_ref[1, t]
            xr = jax.nn.relu(_mm(st * inv, wlv) + blv + _mm(x_ref[t], wrv))
            h1 = _gru(xr, h1, g[0], g[1], g[2], g[3])
            h2 = _gru(h1, h2, g[4], g[5], g[6], g[7])
            h3 = _gru(h2, h3, g[8], g[9], g[10], g[11])
        h1 = _gru(xr, h1, g[0], g[1], g[2], g[3])
        h2 = _gru(h1, h2, g[4], g[5], g[6], g[7])
        h3 = _gru(h2, h3, g[8], g[9], g[10], g[11])
        y1_ref[...] = _mm(h3, pw_ref[...]) + pb_ref[...]
        h1_ref[...] = h1
        h2_ref[...] = h2
        h3_ref[...] = h3

    full2 = lambda a: pl.BlockSpec(a.shape, lambda i: (0, 0))
    out = pl.pallas_call(
        body,
        grid=(_N // _R,),
        in_specs=[
            pl.BlockSpec((4, _R, _C), lambda i: (0, i, 0)),
            pl.BlockSpec((2, 4, _R, _C), lambda i: (0, 0, i, 0)),
            pl.BlockSpec((2, _R, _C), lambda i: (0, i, 0)),
            full2(wl), full2(wr), full2(bl),
            full2(gw[0]), full2(gw[1]), full2(gw[2]), full2(gw[3]),
            full2(gw[4]), full2(gw[5]), full2(gw[6]), full2(gw[7]),
            full2(gw[8]), full2(gw[9]), full2(gw[10]), full2(gw[11]),
            full2(pw), full2(pb),
        ],
        out_specs=[
            pl.BlockSpec((_R, _C), lambda i: (i, 0)),
            pl.BlockSpec((_R, _H), lambda i: (i, 0)),
            pl.BlockSpec((_R, _H), lambda i: (i, 0)),
            pl.BlockSpec((_R, _H), lambda i: (i, 0)),
        ],
        out_shape=[
            jax.ShapeDtypeStruct((_N, _C), jnp.float32),
            jax.ShapeDtypeStruct((_N, _H), jnp.float32),
            jax.ShapeDtypeStruct((_N, _H), jnp.float32),
            jax.ShapeDtypeStruct((_N, _H), jnp.float32),
        ],
    )(x_all, s_parts, cnt_parts, wl, wr, bl, *gw, pw, pb)
    return out


def _tc_dec(y1, s2_parts, cnt_parts, h1, h2, h3, wl, wr, bl, gw, pw, pb):
    def body(y_ref, s_ref, cnt_ref, h1_ref, h2_ref, h3_ref,
             wl_ref, wr_ref, bl_ref,
             wi1, wh1, bi1, bh1, wi2, wh2, bi2, bh2, wi3, wh3, bi3, bh3,
             pw_ref, pb_ref, y2_ref):
        inv = 1.0 / jnp.clip(cnt_ref[0] + cnt_ref[1], 1.0, None)
        st = s_ref[0] + s_ref[1]
        xr = jax.nn.relu(_mm(st * inv, wl_ref[...]) + bl_ref[...]
                         + _mm(y_ref[...], wr_ref[...]))
        h1 = _gru(xr, h1_ref[...], wi1[...], wh1[...], bi1[...], bh1[...])
        h2 = _gru(h1, h2_ref[...], wi2[...], wh2[...], bi2[...], bh2[...])
        h3 = _gru(h2, h3_ref[...], wi3[...], wh3[...], bi3[...], bh3[...])
        y2_ref[...] = _mm(h3, pw_ref[...]) + pb_ref[...]

    full2 = lambda a: pl.BlockSpec(a.shape, lambda i: (0, 0))
    rb = pl.BlockSpec((_R, _C), lambda i: (i, 0))
    return pl.pallas_call(
        body,
        grid=(_N // _R,),
        in_specs=[
            rb,
            pl.BlockSpec((2, _R, _C), lambda i: (0, i, 0)),
            pl.BlockSpec((2, _R, _C), lambda i: (0, i, 0)),
            rb, rb, rb,
            full2(wl), full2(wr), full2(bl),
            full2(gw[0]), full2(gw[1]), full2(gw[2]), full2(gw[3]),
            full2(gw[4]), full2(gw[5]), full2(gw[6]), full2(gw[7]),
            full2(gw[8]), full2(gw[9]), full2(gw[10]), full2(gw[11]),
            full2(pw), full2(pb),
        ],
        out_specs=rb,
        out_shape=jax.ShapeDtypeStruct((_N, _C), jnp.float32),
    )(y1, s2_parts, cnt_parts, h1, h2, h3, wl, wr, bl, *gw, pw, pb)


def kernel(x_seq, edge_index, sage_Wl, sage_bl, sage_Wr,
           g1_Wih, g1_Whh, g1_bih, g1_bhh,
           g2_Wih, g2_Whh, g2_bih, g2_bhh,
           g3_Wih, g3_Whh, g3_bih, g3_bhh,
           proj_W, proj_b):
    b, p, n, c = x_seq.shape
    x_all = x_seq.reshape(p, n, c)

    src = edge_index[0].astype(jnp.int32)
    dst = edge_index[1].astype(jnp.int32)
    npad = _EPAD - _E
    src_r = jnp.concatenate([src, jnp.zeros((npad,), jnp.int32)]).reshape(
        _NW, _NCH, _K)
    dst_r = jnp.concatenate([dst, jnp.full((npad,), _DUMMY, jnp.int32)]).reshape(
        _NW, _NCH, _K)
    zrow = jnp.zeros((_RPT, _C), jnp.float32)
    ones_b = jnp.ones((_K, _C), jnp.float32)

    wl = sage_Wl.T
    wr = sage_Wr.T
    bl = sage_bl.reshape(1, _H)
    gw = (g1_Wih.T, g1_Whh.T, g1_bih.reshape(1, -1), g1_bhh.reshape(1, -1),
          g2_Wih.T, g2_Whh.T, g2_bih.reshape(1, -1), g2_bhh.reshape(1, -1),
          g3_Wih.T, g3_Whh.T, g3_bih.reshape(1, -1), g3_bhh.reshape(1, -1))
    pw = proj_W.T
    pb = proj_b.reshape(1, _C)

    s_parts, cnt_parts = _spmm_enc(
        x_all[0], x_all[1], x_all[2], x_all[3], src_r, dst_r, zrow, ones_b)
    y1, h1, h2, h3 = _tc_enc(x_all, s_parts, cnt_parts, wl, wr, bl, gw, pw, pb)
    s2_parts = _spmm_one(y1, src_r, dst_r, zrow)
    y2 = _tc_dec(y1, s2_parts, cnt_parts, h1, h2, h3, wl, wr, bl, gw, pw, pb)

    return jnp.stack([y1, y2], axis=0).reshape(b, 2, n, c)


# speedup vs baseline: 1.5832x; 1.5832x over previous
import functools

import jax
import jax.numpy as jnp
from jax import lax
from jax.experimental import pallas as pl
from jax.experimental.pallas import tpu as pltpu
from jax.experimental.pallas import tpu_sc as plsc

_N = 10000
_E = 320000
_C = 128
_H = 128
_NC = 2
_NS = 16
_NW = _NC * _NS
_K = 128
_NCH = 80
_EPAD = _NW * _K * _NCH
_NPAD = 10240
_RPT = _NPAD // _NS
_DUMMY = _N + 8

_mesh = plsc.VectorSubcoreMesh(
    core_axis_name="c", subcore_axis_name="s", num_cores=_NC, num_subcores=_NS
)

_SC_SCRATCH = [
    pltpu.VMEM((_NCH, _K), jnp.int32),
    pltpu.VMEM((_NCH, _K), jnp.int32),
    pltpu.VMEM((_K, _C), jnp.float32),
    pltpu.VMEM_SHARED((_NPAD, _C), jnp.float32),
    pltpu.SemaphoreType.DMA,
]


def _spmm_pass(xh, idx_s, idx_d, rows, acc, sem):
    def chunk(j, carry):
        pltpu.async_copy(xh.at[idx_s.at[j]], rows, sem).wait()
        pltpu.sync_copy(rows, acc.at[idx_d.at[j]], add=True)
        return carry

    lax.fori_loop(0, _NCH, chunk, 0)


def _cnt_pass(ones_v, idx_d, acc):
    def chunk(j, carry):
        pltpu.sync_copy(ones_v, acc.at[idx_d.at[j]], add=True)
        return carry

    lax.fori_loop(0, _NCH, chunk, 0)


def _spmm_enc(x0, x1, x2, x3, src_r, dst_r, zrow, ones_b):
    @functools.partial(
        pl.kernel,
        out_type=(
            jax.ShapeDtypeStruct((_NC, 4, _NPAD, _C), jnp.float32),
            jax.ShapeDtypeStruct((_NC, _NPAD, _C), jnp.float32),
        ),
        mesh=_mesh,
        scratch_types=_SC_SCRATCH,
    )
    def k(x0r, x1r, x2r, x3r, srcr, dstr, zr, ones_h, out_s, out_c,
          idx_s, idx_d, rows, acc, sem):
        c = lax.axis_index("c")
        s = lax.axis_index("s")
        wid = c * _NS + s
        r0 = s * _RPT
        pltpu.sync_copy(srcr.at[wid], idx_s)
        pltpu.sync_copy(dstr.at[wid], idx_d)

        pltpu.sync_copy(ones_h, rows)
        pltpu.sync_copy(zr, acc.at[pl.ds(r0, _RPT)])
        plsc.subcore_barrier()
        _cnt_pass(rows, idx_d, acc)
        plsc.subcore_barrier()
        pltpu.sync_copy(acc.at[pl.ds(r0, _RPT)], out_c.at[c].at[pl.ds(r0, _RPT)])
        plsc.subcore_barrier()

        for t, xh in enumerate((x0r, x1r, x2r, x3r)):
            pltpu.sync_copy(zr, acc.at[pl.ds(r0, _RPT)])
            plsc.subcore_barrier()
            _spmm_pass(xh, idx_s, idx_d, rows, acc, sem)
            plsc.subcore_barrier()
            pltpu.sync_copy(
                acc.at[pl.ds(r0, _RPT)], out_s.at[c].at[t].at[pl.ds(r0, _RPT)]
            )
            plsc.subcore_barrier()

    return k(x0, x1, x2, x3, src_r, dst_r, zrow, ones_b)


def _spmm_one(x, src_r, dst_r, zrow):
    @functools.partial(
        pl.kernel,
        out_type=jax.ShapeDtypeStruct((_NC, _NPAD, _C), jnp.float32),
        mesh=_mesh,
        scratch_types=_SC_SCRATCH,
    )
    def k(xh, srcr, dstr, zr, out_s, idx_s, idx_d, rows, acc, sem):
        c = lax.axis_index("c")
        s = lax.axis_index("s")
        wid = c * _NS + s
        r0 = s * _RPT
        pltpu.sync_copy(srcr.at[wid], idx_s)
        pltpu.sync_copy(dstr.at[wid], idx_d)
        pltpu.sync_copy(zr, acc.at[pl.ds(r0, _RPT)])
        plsc.subcore_barrier()
        _spmm_pass(xh, idx_s, idx_d, rows, acc, sem)
        plsc.subcore_barrier()
        pltpu.sync_copy(acc.at[pl.ds(r0, _RPT)], out_s.at[c].at[pl.ds(r0, _RPT)])

    return k(x, src_r, dst_r, zrow)


def _mm(a, b):
    return jnp.dot(a, b, preferred_element_type=jnp.float32)


def _gru(x, h, wi, wh, bi, bh):
    gi = _mm(x, wi) + bi
    gh = _mm(h, wh) + bh
    r = jax.nn.sigmoid(gi[:, :_H] + gh[:, :_H])
    z = jax.nn.sigmoid(gi[:, _H:2 * _H] + gh[:, _H:2 * _H])
    n = jnp.tanh(gi[:, 2 * _H:] + r * gh[:, 2 * _H:])
    return (1.0 - z) * n + z * h


_R = 1000


def _tc_enc(x_all, s_parts, cnt_parts, wl, wr, bl, gw, pw, pb):
    def body(x_ref, s_ref, cnt_ref, wl_ref, wr_ref, bl_ref,
             wi1, wh1, bi1, bh1, wi2, wh2, bi2, bh2, wi3, wh3, bi3, bh3,
             pw_ref, pb_ref, y1_ref, h1_ref, h2_ref, h3_ref):
        inv = 1.0 / jnp.clip(cnt_ref[0] + cnt_ref[1], 1.0, None)
        wlv, wrv, blv = wl_ref[...], wr_ref[...], bl_ref[...]
        g = (wi1[...], wh1[...], bi1[...], bh1[...],
             wi2[...], wh2[...], bi2[...], bh2[...],
             wi3[...], wh3[...], bi3[...], bh3[...])
        h1 = jnp.zeros((_R, _H), jnp.float32)
        h2 = jnp.zeros((_R, _H), jnp.float32)
        h3 = jnp.zeros((_R, _H), jnp.float32)
        xr = None
        for t in range(4):
            st = s_ref[0, t] + s_ref[1, t]
            xr = jax.nn.relu(_mm(st * inv, wlv) + blv + _mm(x_ref[t], wrv))
            h1 = _gru(xr, h1, g[0], g[1], g[2], g[3])
            h2 = _gru(h1, h2, g[4], g[5], g[6], g[7])
            h3 = _gru(h2, h3, g[8], g[9], g[10], g[11])
        h1 = _gru(xr, h1, g[0], g[1], g[2], g[3])
        h2 = _gru(h1, h2, g[4], g[5], g[6], g[7])
        h3 = _gru(h2, h3, g[8], g[9], g[10], g[11])
        y1_ref[...] = _mm(h3, pw_ref[...]) + pb_ref[...]
        h1_ref[...] = h1
        h2_ref[...] = h2
        h3_ref[...] = h3

    full2 = lambda a: pl.BlockSpec(a.shape, lambda i: (0, 0))
    out = pl.pallas_call(
        body,
        grid=(_N // _R,),
        in_specs=[
            pl.BlockSpec((4, _R, _C), lambda i: (0, i, 0)),
            pl.BlockSpec((2, 4, _R, _C), lambda i: (0, 0, i, 0)),
            pl.BlockSpec((2, _R, _C), lambda i: (0, i, 0)),
            full2(wl), full2(wr), full2(bl),
            full2(gw[0]), full2(gw[1]), full2(gw[2]), full2(gw[3]),
            full2(gw[4]), full2(gw[5]), full2(gw[6]), full2(gw[7]),
            full2(gw[8]), full2(gw[9]), full2(gw[10]), full2(gw[11]),
            full2(pw), full2(pb),
        ],
        out_specs=[
            pl.BlockSpec((_R, _C), lambda i: (i, 0)),
            pl.BlockSpec((_R, _H), lambda i: (i, 0)),
            pl.BlockSpec((_R, _H), lambda i: (i, 0)),
            pl.BlockSpec((_R, _H), lambda i: (i, 0)),
        ],
        out_shape=[
            jax.ShapeDtypeStruct((_N, _C), jnp.float32),
            jax.ShapeDtypeStruct((_N, _H), jnp.float32),
            jax.ShapeDtypeStruct((_N, _H), jnp.float32),
            jax.ShapeDtypeStruct((_N, _H), jnp.float32),
        ],
    )(x_all, s_parts, cnt_parts, wl, wr, bl, *gw, pw, pb)
    return out


def _tc_dec(y1, s2_parts, cnt_parts, h1, h2, h3, wl, wr, bl, gw, pw, pb):
    def body(y_ref, s_ref, cnt_ref, h1_ref, h2_ref, h3_ref,
             wl_ref, wr_ref, bl_ref,
             wi1, wh1, bi1, bh1, wi2, wh2, bi2, bh2, wi3, wh3, bi3, bh3,
             pw_ref, pb_ref, y2_ref):
        inv = 1.0 / jnp.clip(cnt_ref[0] + cnt_ref[1], 1.0, None)
        st = s_ref[0] + s_ref[1]
        xr = jax.nn.relu(_mm(st * inv, wl_ref[...]) + bl_ref[...]
                         + _mm(y_ref[...], wr_ref[...]))
        h1 = _gru(xr, h1_ref[...], wi1[...], wh1[...], bi1[...], bh1[...])
        h2 = _gru(h1, h2_ref[...], wi2[...], wh2[...], bi2[...], bh2[...])
        h3 = _gru(h2, h3_ref[...], wi3[...], wh3[...], bi3[...], bh3[...])
        y2_ref[...] = _mm(h3, pw_ref[...]) + pb_ref[...]

    full2 = lambda a: pl.BlockSpec(a.shape, lambda i: (0, 0))
    rb = pl.BlockSpec((_R, _C), lambda i: (i, 0))
    return pl.pallas_call(
        body,
        grid=(_N // _R,),
        in_specs=[
            rb,
            pl.BlockSpec((2, _R, _C), lambda i: (0, i, 0)),
            pl.BlockSpec((2, _R, _C), lambda i: (0, i, 0)),
            rb, rb, rb,
            full2(wl), full2(wr), full2(bl),
            full2(gw[0]), full2(gw[1]), full2(gw[2]), full2(gw[3]),
            full2(gw[4]), full2(gw[5]), full2(gw[6]), full2(gw[7]),
            full2(gw[8]), full2(gw[9]), full2(gw[10]), full2(gw[11]),
            full2(pw), full2(pb),
        ],
        out_specs=rb,
        out_shape=jax.ShapeDtypeStruct((_N, _C), jnp.float32),
    )(y1, s2_parts, cnt_parts, h1, h2, h3, wl, wr, bl, *gw, pw, pb)


def kernel(x_seq, edge_index, sage_Wl, sage_bl, sage_Wr,
           g1_Wih, g1_Whh, g1_bih, g1_bhh,
           g2_Wih, g2_Whh, g2_bih, g2_bhh,
           g3_Wih, g3_Whh, g3_bih, g3_bhh,
           proj_W, proj_b):
    b, p, n, c = x_seq.shape
    x_all = x_seq.reshape(p, n, c)

    src = edge_index[0].astype(jnp.int32)
    dst = edge_index[1].astype(jnp.int32)
    npad = _EPAD - _E
    src_r = jnp.concatenate([src, jnp.zeros((npad,), jnp.int32)]).reshape(
        _NW, _NCH, _K)
    dst_r = jnp.concatenate([dst, jnp.full((npad,), _DUMMY, jnp.int32)]).reshape(
        _NW, _NCH, _K)
    zrow = jnp.zeros((_RPT, _C), jnp.float32)
    ones_b = jnp.ones((_K, _C), jnp.float32)

    wl = sage_Wl.T
    wr = sage_Wr.T
    bl = sage_bl.reshape(1, _H)
    gw = (g1_Wih.T, g1_Whh.T, g1_bih.reshape(1, -1), g1_bhh.reshape(1, -1),
          g2_Wih.T, g2_Whh.T, g2_bih.reshape(1, -1), g2_bhh.reshape(1, -1),
          g3_Wih.T, g3_Whh.T, g3_bih.reshape(1, -1), g3_bhh.reshape(1, -1))
    pw = proj_W.T
    pb = proj_b.reshape(1, _C)

    s_parts, cnt_parts = _spmm_enc(
        x_all[0], x_all[1], x_all[2], x_all[3], src_r, dst_r, zrow, ones_b)
    y1, h1, h2, h3 = _tc_enc(x_all, s_parts, cnt_parts, wl, wr, bl, gw, pw, pb)
    s2_parts = _spmm_one(y1, src_r, dst_r, zrow)
    y2 = _tc_dec(y1, s2_parts, cnt_parts, h1, h2, h3, wl, wr, bl, gw, pw, pb)

    return jnp.stack([y1, y2], axis=0).reshape(b, 2, n, c)
